# baseline (speedup 1.0000x reference)
import functools

import jax
import jax.numpy as jnp
from jax import lax
from jax.experimental import pallas as pl
from jax.experimental.pallas import tpu as pltpu
from jax.experimental.pallas import tpu_sc as plsc

NC = 2
NS = 16


def _sc_mesh():
    return plsc.VectorSubcoreMesh(
        core_axis_name="c", subcore_axis_name="s", num_cores=NC, num_subcores=NS
    )


def _zero_phase(zeros_hbm, bounce, acc, s, rb, nblk, nrb):
    pltpu.sync_copy(zeros_hbm, bounce)
    for j in range(nrb):
        bid = s * nrb + j

        @pl.when(bid < nblk)
        def _():
            pltpu.sync_copy(bounce, acc.at[pl.ds(bid * rb, rb)])


def _copy_out_phase(acc, bounce, out_hbm, c, s, n, rb, nblk, nrb):
    for j in range(nrb):
        bid = s * nrb + j

        @pl.when(bid < nblk)
        def _():
            pltpu.sync_copy(acc.at[pl.ds(bid * rb, rb)], bounce)
            pltpu.sync_copy(bounce, out_hbm.at[pl.ds(c * n + bid * rb, rb)])


def _make_deg(n, e):
    ch = 40
    ew = e // (NC * NS)
    nch = ew // ch
    npair = nch // 2
    rb = 80
    nblk = n // rb
    nrb = -(-nblk // NS)

    @functools.partial(
        pl.kernel,
        out_type=jax.ShapeDtypeStruct((NC * n, 128), jnp.float32),
        mesh=_sc_mesh(),
        scratch_types=[
            pltpu.VMEM((ch,), jnp.int32),
            pltpu.VMEM((ch,), jnp.int32),
            pltpu.VMEM((ch, 128), jnp.float32),
            pltpu.VMEM((rb, 128), jnp.float32),
            pltpu.VMEM_SHARED((n, 128), jnp.float32),
            pltpu.SemaphoreType.DMA,
            pltpu.SemaphoreType.DMA,
        ],
    )
    def deg_kernel(dst_hbm, ones_hbm, zeros_hbm, out_hbm,
                   idxv0, idxv1, onesv, bounce, acc, sem0, sem1):
        c = lax.axis_index("c")
        s = lax.axis_index("s")
        idxv = (idxv0, idxv1)
        sem = (sem0, sem1)
        _zero_phase(zeros_hbm, bounce, acc, s, rb, nblk, nrb)
        pltpu.sync_copy(ones_hbm, onesv)
        plsc.subcore_barrier()
        e0 = c * (e // 2) + s * ew

        def pair(k, carry):
            for b in range(2):
                j = 2 * k + b

                @pl.when(k > 0)
                def _():
                    pltpu.make_async_copy(onesv, acc.at[idxv[b]], sem[b]).wait()

                pltpu.sync_copy(dst_hbm.at[pl.ds(e0 + j * ch, ch)], idxv[b])
                pltpu.async_copy(onesv, acc.at[idxv[b]], sem[b], add=True)
            return carry

        lax.fori_loop(0, npair, pair, 0)
        for b in range(2):
            pltpu.make_async_copy(onesv, acc.at[idxv[b]], sem[b]).wait()
        plsc.subcore_barrier()
        _copy_out_phase(acc, bounce, out_hbm, c, s, n, rb, nblk, nrb)

    return deg_kernel


def _make_spmm_edge(n, e, f):
    ch = 40
    ew = e // (NC * NS)
    nch = ew // ch
    npair = nch // 2
    rb = 80
    nblk = n // rb
    nrb = -(-nblk // NS)

    @functools.partial(
        pl.kernel,
        out_type=jax.ShapeDtypeStruct((NC * n, f), jnp.float32),
        mesh=_sc_mesh(),
        scratch_types=[
            pltpu.VMEM((ch,), jnp.int32),
            pltpu.VMEM((ch,), jnp.int32),
            pltpu.VMEM((ch,), jnp.int32),
            pltpu.VMEM((ch, f), jnp.float32),
            pltpu.VMEM((ch, f), jnp.float32),
            pltpu.VMEM((rb, f), jnp.float32),
            pltpu.VMEM_SHARED((n, f), jnp.float32),
            pltpu.SemaphoreType.DMA,
            pltpu.SemaphoreType.DMA,
        ],
    )
    def spmm_kernel(src_hbm, dst_hbm, g_hbm, zeros_hbm, out_hbm,
                    srcv, dstv0, dstv1, rows0, rows1, bounce, acc, sem0, sem1):
        c = lax.axis_index("c")
        s = lax.axis_index("s")
        dstv = (dstv0, dstv1)
        rows = (rows0, rows1)
        sem = (sem0, sem1)
        _zero_phase(zeros_hbm, bounce, acc, s, rb, nblk, nrb)
        plsc.subcore_barrier()
        e0 = c * (e // 2) + s * ew

        def pair(k, carry):
            for b in range(2):
                j = 2 * k + b
                base = e0 + j * ch
                pltpu.sync_copy(src_hbm.at[pl.ds(base, ch)], srcv)

                @pl.when(k > 0)
                def _():
                    pltpu.make_async_copy(rows[b], acc.at[dstv[b]], sem[b]).wait()

                pltpu.sync_copy(g_hbm.at[srcv], rows[b])
                pltpu.sync_copy(dst_hbm.at[pl.ds(base, ch)], dstv[b])
                pltpu.async_copy(rows[b], acc.at[dstv[b]], sem[b], add=True)
            return carry

        lax.fori_loop(0, npair, pair, 0)
        for b in range(2):
            pltpu.make_async_copy(rows[b], acc.at[dstv[b]], sem[b]).wait()
        plsc.subcore_barrier()
        _copy_out_phase(acc, bounce, out_hbm, c, s, n, rb, nblk, nrb)

    return spmm_kernel


def _make_spmm_feat(n, e, fh):
    ch = 80
    ew = e // NS
    nch = ew // ch
    npair = nch // 2
    rb = 80
    nblk = n // rb
    nrb = -(-nblk // NS)

    @functools.partial(
        pl.kernel,
        out_type=jax.ShapeDtypeStruct((NC * n, fh), jnp.float32),
        mesh=_sc_mesh(),
        scratch_types=[
            pltpu.VMEM((ch,), jnp.int32),
            pltpu.VMEM((ch,), jnp.int32),
            pltpu.VMEM((ch,), jnp.int32),
            pltpu.VMEM((ch,), jnp.int32),
            pltpu.VMEM((ch, fh), jnp.float32),
            pltpu.VMEM((ch, fh), jnp.float32),
            pltpu.VMEM((rb, fh), jnp.float32),
            pltpu.VMEM_SHARED((n, fh), jnp.float32),
            pltpu.SemaphoreType.DMA,
            pltpu.SemaphoreType.DMA,
        ],
    )
    def spmm_kernel(src_hbm, dst_hbm, g_hbm, zeros_hbm, out_hbm,
                    srcv, idxg, dstv0, dstv1, rows0, rows1, bounce, acc,
                    sem0, sem1):
        c = lax.axis_index("c")
        s = lax.axis_index("s")
        dstv = (dstv0, dstv1)
        rows = (rows0, rows1)
        sem = (sem0, sem1)
        _zero_phase(zeros_hbm, bounce, acc, s, rb, nblk, nrb)
        plsc.subcore_barrier()
        e0 = s * ew

        def pair(k, carry):
            for b in range(2):
                j = 2 * k + b
                base = e0 + j * ch
                pltpu.sync_copy(src_hbm.at[pl.ds(base, ch)], srcv)
                for kk in range(ch // 16):
                    sl = pl.ds(kk * 16, 16)
                    idxg[sl] = srcv[sl] * 2 + c

                @pl.when(k > 0)
                def _():
                    pltpu.make_async_copy(rows[b], acc.at[dstv[b]], sem[b]).wait()

                pltpu.sync_copy(g_hbm.at[idxg], rows[b])
                pltpu.sync_copy(dst_hbm.at[pl.ds(base, ch)], dstv[b])
                pltpu.async_copy(rows[b], acc.at[dstv[b]], sem[b], add=True)
            return carry

        lax.fori_loop(0, npair, pair, 0)
        for b in range(2):
            pltpu.make_async_copy(rows[b], acc.at[dstv[b]], sem[b]).wait()
        plsc.subcore_barrier()
        _copy_out_phase(acc, bounce, out_hbm, c, s, n, rb, nblk, nrb)

    return spmm_kernel


def _p1_kernel(dega_ref, degb_ref, x_ref, g1_ref, d16_ref):
    deg = dega_ref[...][:, :1] + degb_ref[...][:, :1] + 1.0
    d = lax.rsqrt(deg)
    g1_ref[...] = x_ref[...] * d
    d16_ref[...] = jnp.broadcast_to(d, d16_ref.shape)


def _p2_kernel(s1a_ref, s1b_ref, g1_ref, d16_ref, w1t_ref, b1_ref, g2_ref):
    d = d16_ref[...][:, :1]
    u = (s1a_ref[...] + s1b_ref[...] + g1_ref[...]) * d
    h = (jnp.dot(u, w1t_ref[...], preferred_element_type=jnp.float32)
         + b1_ref[...])
    g2_ref[...] = jnp.maximum(h, 0.0) * d


def _p3_kernel(s2a_ref, s2b_ref, g2_ref, d16_ref, w2at_ref, w2bt_ref, b2_ref,
               wf1t_ref, bf1_ref, wf2t_ref, bf2_ref, out_ref):
    d = d16_ref[...][:, :1]
    g2 = g2_ref[...]
    u = (s2a_ref[...] + g2[:, :128]) * d
    v = (s2b_ref[...] + g2[:, 128:]) * d
    h2 = jnp.maximum(
        jnp.dot(u, w2at_ref[...], preferred_element_type=jnp.float32)
        + jnp.dot(v, w2bt_ref[...], preferred_element_type=jnp.float32)
        + b2_ref[...], 0.0)
    h3 = jnp.maximum(
        jnp.dot(h2, wf1t_ref[...], preferred_element_type=jnp.float32)
        + bf1_ref[...], 0.0)
    out_ref[...] = (jnp.dot(h3, wf2t_ref[...], preferred_element_type=jnp.float32)
                    + bf2_ref[...])


def _row_spec(blk, width):
    return pl.BlockSpec((blk, width), lambda i: (i, 0))


def _full_spec(shape):
    return pl.BlockSpec(shape, lambda i: tuple(0 for _ in shape))


def kernel(x, edge_index, W1, b1, W2, b2, Wf1, bf1, Wf2, bf2):
    n, nfeat = x.shape
    e = edge_index.shape[1]
    nhid = W1.shape[0]
    blk = 1000
    grid = (n // blk,)

    src = edge_index[0]
    dst = edge_index[1]

    deg2 = _make_deg(n, e)(
        dst,
        jnp.ones((40, 128), jnp.float32),
        jnp.zeros((80, 128), jnp.float32),
    )

    g1, d16 = pl.pallas_call(
        _p1_kernel,
        grid=grid,
        in_specs=[_row_spec(blk, 128), _row_spec(blk, 128), _row_spec(blk, nfeat)],
        out_specs=[_row_spec(blk, nfeat), _row_spec(blk, 16)],
        out_shape=[
            jax.ShapeDtypeStruct((n, nfeat), jnp.float32),
            jax.ShapeDtypeStruct((n, 16), jnp.float32),
        ],
    )(deg2[:n], deg2[n:], x)

    zeros128a = jnp.zeros((80, nfeat), jnp.float32)
    s1 = _make_spmm_edge(n, e, nfeat)(src, dst, g1, zeros128a)

    g2 = pl.pallas_call(
        _p2_kernel,
        grid=grid,
        in_specs=[
            _row_spec(blk, nfeat), _row_spec(blk, nfeat), _row_spec(blk, nfeat),
            _row_spec(blk, 16),
            _full_spec((nfeat, nhid)), _full_spec((1, nhid)),
        ],
        out_specs=_row_spec(blk, nhid),
        out_shape=jax.ShapeDtypeStruct((n, nhid), jnp.float32),
    )(s1[:n], s1[n:], g1, d16, W1.T, b1.reshape(1, nhid))

    zeros128 = jnp.zeros((80, nhid // 2), jnp.float32)
    s2 = _make_spmm_feat(n, e, nhid // 2)(src, dst, g2.reshape(2 * n, nhid // 2),
                                          zeros128)

    out = pl.pallas_call(
        _p3_kernel,
        grid=grid,
        in_specs=[
            _row_spec(blk, 128), _row_spec(blk, 128), _row_spec(blk, nhid),
            _row_spec(blk, 16),
            _full_spec((128, nhid)), _full_spec((128, nhid)), _full_spec((1, nhid)),
            _full_spec((nhid, 128)), _full_spec((1, 128)),
            _full_spec((128, 16)), _full_spec((1, 16)),
        ],
        out_specs=_row_spec(blk, 16),
        out_shape=jax.ShapeDtypeStruct((n, 16), jnp.float32),
    )(s2[:n], s2[n:], g2, d16,
      W2[:, :128].T, W2[:, 128:].T, b2.reshape(1, nhid),
      Wf1.T, bf1.reshape(1, 128),
      Wf2.T, bf2.reshape(1, 16))
    return out


# speedup vs baseline: 1.2373x; 1.2373x over previous
import functools

import jax
import jax.numpy as jnp
from jax import lax
from jax.experimental import pallas as pl
from jax.experimental.pallas import tpu as pltpu
from jax.experimental.pallas import tpu_sc as plsc

NC = 2
NS = 16


def _sc_mesh():
    return plsc.VectorSubcoreMesh(
        core_axis_name="c", subcore_axis_name="s", num_cores=NC, num_subcores=NS
    )


def _zero_phase(zeros_hbm, bounce, acc, s, rb, nblk, nrb):
    pltpu.sync_copy(zeros_hbm, bounce)
    for j in range(nrb):
        bid = s * nrb + j

        @pl.when(bid < nblk)
        def _():
            pltpu.sync_copy(bounce, acc.at[pl.ds(bid * rb, rb)])


def _copy_out_phase(acc, bounce, out_hbm, c, s, n, rb, nblk, nrb):
    for j in range(nrb):
        bid = s * nrb + j

        @pl.when(bid < nblk)
        def _():
            pltpu.sync_copy(acc.at[pl.ds(bid * rb, rb)], bounce)
            pltpu.sync_copy(bounce, out_hbm.at[pl.ds(c * n + bid * rb, rb)])


def _make_deg(n, e):
    ch = 80
    ew = e // (NC * NS)
    nch = ew // ch
    npair = nch // 2
    tail = nch - 2 * npair
    rb = 80
    nblk = n // rb
    nrb = -(-nblk // NS)

    @functools.partial(
        pl.kernel,
        out_type=jax.ShapeDtypeStruct((NC * n, 128), jnp.float32),
        mesh=_sc_mesh(),
        scratch_types=[
            pltpu.VMEM((ch,), jnp.int32),
            pltpu.VMEM((ch,), jnp.int32),
            pltpu.VMEM((ch, 128), jnp.float32),
            pltpu.VMEM((rb, 128), jnp.float32),
            pltpu.VMEM_SHARED((n, 128), jnp.float32),
            pltpu.SemaphoreType.DMA,
            pltpu.SemaphoreType.DMA,
        ],
    )
    def deg_kernel(dst_hbm, ones_hbm, zeros_hbm, out_hbm,
                   idxv0, idxv1, onesv, bounce, acc, sem0, sem1):
        c = lax.axis_index("c")
        s = lax.axis_index("s")
        idxv = (idxv0, idxv1)
        sem = (sem0, sem1)
        _zero_phase(zeros_hbm, bounce, acc, s, rb, nblk, nrb)
        pltpu.sync_copy(ones_hbm, onesv)
        plsc.subcore_barrier()
        e0 = c * (e // 2) + s * ew

        def pair(k, carry):
            for b in range(2):
                j = 2 * k + b

                @pl.when(k > 0)
                def _():
                    pltpu.make_async_copy(onesv, acc.at[idxv[b]], sem[b]).wait()

                pltpu.sync_copy(dst_hbm.at[pl.ds(e0 + j * ch, ch)], idxv[b])
                pltpu.async_copy(onesv, acc.at[idxv[b]], sem[b], add=True)
            return carry

        lax.fori_loop(0, npair, pair, 0)
        if tail:
            if npair > 0:
                pltpu.make_async_copy(onesv, acc.at[idxv[0]], sem[0]).wait()
            pltpu.sync_copy(dst_hbm.at[pl.ds(e0 + 2 * npair * ch, ch)], idxv[0])
            pltpu.async_copy(onesv, acc.at[idxv[0]], sem[0], add=True)
        pltpu.make_async_copy(onesv, acc.at[idxv[0]], sem[0]).wait()
        if npair > 0:
            pltpu.make_async_copy(onesv, acc.at[idxv[1]], sem[1]).wait()
        plsc.subcore_barrier()
        _copy_out_phase(acc, bounce, out_hbm, c, s, n, rb, nblk, nrb)

    return deg_kernel


def _make_spmm_edge(n, e, f):
    ch = 80
    ew = e // (NC * NS)
    nch = ew // ch
    npair = nch // 2
    tail = nch - 2 * npair
    rb = 80
    nblk = n // rb
    nrb = -(-nblk // NS)

    @functools.partial(
        pl.kernel,
        out_type=jax.ShapeDtypeStruct((NC * n, f), jnp.float32),
        mesh=_sc_mesh(),
        scratch_types=[
            pltpu.VMEM((ch,), jnp.int32),
            pltpu.VMEM((ch,), jnp.int32),
            pltpu.VMEM((ch,), jnp.int32),
            pltpu.VMEM((ch, f), jnp.float32),
            pltpu.VMEM((ch, f), jnp.float32),
            pltpu.VMEM((rb, f), jnp.float32),
            pltpu.VMEM_SHARED((n, f), jnp.float32),
            pltpu.SemaphoreType.DMA,
            pltpu.SemaphoreType.DMA,
        ],
    )
    def spmm_kernel(src_hbm, dst_hbm, g_hbm, zeros_hbm, out_hbm,
                    srcv, dstv0, dstv1, rows0, rows1, bounce, acc, sem0, sem1):
        c = lax.axis_index("c")
        s = lax.axis_index("s")
        dstv = (dstv0, dstv1)
        rows = (rows0, rows1)
        sem = (sem0, sem1)
        _zero_phase(zeros_hbm, bounce, acc, s, rb, nblk, nrb)
        plsc.subcore_barrier()
        e0 = c * (e // 2) + s * ew

        def pair(k, carry):
            for b in range(2):
                j = 2 * k + b
                base = e0 + j * ch
                pltpu.sync_copy(src_hbm.at[pl.ds(base, ch)], srcv)

                @pl.when(k > 0)
                def _():
                    pltpu.make_async_copy(rows[b], acc.at[dstv[b]], sem[b]).wait()

                pltpu.sync_copy(g_hbm.at[srcv], rows[b])
                pltpu.sync_copy(dst_hbm.at[pl.ds(base, ch)], dstv[b])
                pltpu.async_copy(rows[b], acc.at[dstv[b]], sem[b], add=True)
            return carry

        lax.fori_loop(0, npair, pair, 0)
        if tail:
            base = e0 + 2 * npair * ch
            pltpu.sync_copy(src_hbm.at[pl.ds(base, ch)], srcv)
            if npair > 0:
                pltpu.make_async_copy(rows[0], acc.at[dstv[0]], sem[0]).wait()
            pltpu.sync_copy(g_hbm.at[srcv], rows[0])
            pltpu.sync_copy(dst_hbm.at[pl.ds(base, ch)], dstv[0])
            pltpu.async_copy(rows[0], acc.at[dstv[0]], sem[0], add=True)
        pltpu.make_async_copy(rows[0], acc.at[dstv[0]], sem[0]).wait()
        if npair > 0:
            pltpu.make_async_copy(rows[1], acc.at[dstv[1]], sem[1]).wait()
        plsc.subcore_barrier()
        _copy_out_phase(acc, bounce, out_hbm, c, s, n, rb, nblk, nrb)

    return spmm_kernel


def _make_spmm_feat(n, e, fh):
    ch = 80
    ew = e // NS
    nch = ew // ch
    npair = nch // 2
    rb = 80
    nblk = n // rb
    nrb = -(-nblk // NS)

    @functools.partial(
        pl.kernel,
        out_type=jax.ShapeDtypeStruct((NC * n, fh), jnp.float32),
        mesh=_sc_mesh(),
        scratch_types=[
            pltpu.VMEM((ch,), jnp.int32),
            pltpu.VMEM((ch,), jnp.int32),
            pltpu.VMEM((ch,), jnp.int32),
            pltpu.VMEM((ch,), jnp.int32),
            pltpu.VMEM((ch, fh), jnp.float32),
            pltpu.VMEM((ch, fh), jnp.float32),
            pltpu.VMEM((rb, fh), jnp.float32),
            pltpu.VMEM_SHARED((n, fh), jnp.float32),
            pltpu.SemaphoreType.DMA,
            pltpu.SemaphoreType.DMA,
        ],
    )
    def spmm_kernel(src_hbm, dst_hbm, g_hbm, zeros_hbm, out_hbm,
                    srcv, idxg, dstv0, dstv1, rows0, rows1, bounce, acc,
                    sem0, sem1):
        c = lax.axis_index("c")
        s = lax.axis_index("s")
        dstv = (dstv0, dstv1)
        rows = (rows0, rows1)
        sem = (sem0, sem1)
        _zero_phase(zeros_hbm, bounce, acc, s, rb, nblk, nrb)
        plsc.subcore_barrier()
        e0 = s * ew

        def pair(k, carry):
            for b in range(2):
                j = 2 * k + b
                base = e0 + j * ch
                pltpu.sync_copy(src_hbm.at[pl.ds(base, ch)], srcv)
                for kk in range(ch // 16):
                    sl = pl.ds(kk * 16, 16)
                    idxg[sl] = srcv[sl] * 2 + c

                @pl.when(k > 0)
                def _():
                    pltpu.make_async_copy(rows[b], acc.at[dstv[b]], sem[b]).wait()

                pltpu.sync_copy(g_hbm.at[idxg], rows[b])
                pltpu.sync_copy(dst_hbm.at[pl.ds(base, ch)], dstv[b])
                pltpu.async_copy(rows[b], acc.at[dstv[b]], sem[b], add=True)
            return carry

        lax.fori_loop(0, npair, pair, 0)
        for b in range(2):
            pltpu.make_async_copy(rows[b], acc.at[dstv[b]], sem[b]).wait()
        plsc.subcore_barrier()
        _copy_out_phase(acc, bounce, out_hbm, c, s, n, rb, nblk, nrb)

    return spmm_kernel


def _p1_kernel(dega_ref, degb_ref, x_ref, g1_ref, d16_ref):
    deg = dega_ref[...][:, :1] + degb_ref[...][:, :1] + 1.0
    d = lax.rsqrt(deg)
    g1_ref[...] = x_ref[...] * d
    d16_ref[...] = jnp.broadcast_to(d, d16_ref.shape)


def _p2_kernel(s1a_ref, s1b_ref, g1_ref, d16_ref, w1t_ref, b1_ref, g2_ref):
    d = d16_ref[...][:, :1]
    u = (s1a_ref[...] + s1b_ref[...] + g1_ref[...]) * d
    h = (jnp.dot(u, w1t_ref[...], preferred_element_type=jnp.float32)
         + b1_ref[...])
    g2_ref[...] = jnp.maximum(h, 0.0) * d


def _p3_kernel(s2a_ref, s2b_ref, g2_ref, d16_ref, w2at_ref, w2bt_ref, b2_ref,
               wf1t_ref, bf1_ref, wf2t_ref, bf2_ref, out_ref):
    d = d16_ref[...][:, :1]
    g2 = g2_ref[...]
    u = (s2a_ref[...] + g2[:, :128]) * d
    v = (s2b_ref[...] + g2[:, 128:]) * d
    h2 = jnp.maximum(
        jnp.dot(u, w2at_ref[...], preferred_element_type=jnp.float32)
        + jnp.dot(v, w2bt_ref[...], preferred_element_type=jnp.float32)
        + b2_ref[...], 0.0)
    h3 = jnp.maximum(
        jnp.dot(h2, wf1t_ref[...], preferred_element_type=jnp.float32)
        + bf1_ref[...], 0.0)
    out_ref[...] = (jnp.dot(h3, wf2t_ref[...], preferred_element_type=jnp.float32)
                    + bf2_ref[...])


def _row_spec(blk, width):
    return pl.BlockSpec((blk, width), lambda i: (i, 0))


def _full_spec(shape):
    return pl.BlockSpec(shape, lambda i: tuple(0 for _ in shape))


def kernel(x, edge_index, W1, b1, W2, b2, Wf1, bf1, Wf2, bf2):
    n, nfeat = x.shape
    e = edge_index.shape[1]
    nhid = W1.shape[0]
    blk = 1000
    grid = (n // blk,)

    src = edge_index[0]
    dst = edge_index[1]

    deg2 = _make_deg(n, e)(
        dst,
        jnp.ones((80, 128), jnp.float32),
        jnp.zeros((80, 128), jnp.float32),
    )

    g1, d16 = pl.pallas_call(
        _p1_kernel,
        grid=grid,
        in_specs=[_row_spec(blk, 128), _row_spec(blk, 128), _row_spec(blk, nfeat)],
        out_specs=[_row_spec(blk, nfeat), _row_spec(blk, 16)],
        out_shape=[
            jax.ShapeDtypeStruct((n, nfeat), jnp.float32),
            jax.ShapeDtypeStruct((n, 16), jnp.float32),
        ],
    )(deg2[:n], deg2[n:], x)

    zeros128a = jnp.zeros((80, nfeat), jnp.float32)
    s1 = _make_spmm_edge(n, e, nfeat)(src, dst, g1, zeros128a)

    g2 = pl.pallas_call(
        _p2_kernel,
        grid=grid,
        in_specs=[
            _row_spec(blk, nfeat), _row_spec(blk, nfeat), _row_spec(blk, nfeat),
            _row_spec(blk, 16),
            _full_spec((nfeat, nhid)), _full_spec((1, nhid)),
        ],
        out_specs=_row_spec(blk, nhid),
        out_shape=jax.ShapeDtypeStruct((n, nhid), jnp.float32),
    )(s1[:n], s1[n:], g1, d16, W1.T, b1.reshape(1, nhid))

    zeros128 = jnp.zeros((80, nhid // 2), jnp.float32)
    s2 = _make_spmm_feat(n, e, nhid // 2)(src, dst, g2.reshape(2 * n, nhid // 2),
                                          zeros128)

    out = pl.pallas_call(
        _p3_kernel,
        grid=grid,
        in_specs=[
            _row_spec(blk, 128), _row_spec(blk, 128), _row_spec(blk, nhid),
            _row_spec(blk, 16),
            _full_spec((128, nhid)), _full_spec((128, nhid)), _full_spec((1, nhid)),
            _full_spec((nhid, 128)), _full_spec((1, 128)),
            _full_spec((128, 16)), _full_spec((1, 16)),
        ],
        out_specs=_row_spec(blk, 16),
        out_shape=jax.ShapeDtypeStruct((n, 16), jnp.float32),
    )(s2[:n], s2[n:], g2, d16,
      W2[:, :128].T, W2[:, 128:].T, b2.reshape(1, nhid),
      Wf1.T, bf1.reshape(1, 128),
      Wf2.T, bf2.reshape(1, 16))
    return out


# speedup vs baseline: 1.5393x; 1.2441x over previous
import functools

import jax
import jax.numpy as jnp
from jax import lax
from jax.experimental import pallas as pl
from jax.experimental.pallas import tpu as pltpu
from jax.experimental.pallas import tpu_sc as plsc

NC = 2
NS = 16


def _sc_mesh():
    return plsc.VectorSubcoreMesh(
        core_axis_name="c", subcore_axis_name="s", num_cores=NC, num_subcores=NS
    )


def _zero_phase(zeros_hbm, bounce, acc, s, rb, nblk, nrb):
    pltpu.sync_copy(zeros_hbm, bounce)
    for j in range(nrb):
        bid = s * nrb + j

        @pl.when(bid < nblk)
        def _():
            pltpu.sync_copy(bounce, acc.at[pl.ds(bid * rb, rb)])


def _copy_out_phase(acc, bounce, out_hbm, c, s, n, rb, nblk, nrb):
    for j in range(nrb):
        bid = s * nrb + j

        @pl.when(bid < nblk)
        def _():
            pltpu.sync_copy(acc.at[pl.ds(bid * rb, rb)], bounce)
            pltpu.sync_copy(bounce, out_hbm.at[pl.ds(c * n + bid * rb, rb)])


def _make_deg(n, e):
    ch = 128
    tot = (e // NC) // ch
    ncs = tot // NS
    rem = tot % NS
    assert ncs % 2 == 0
    npair = ncs // 2
    rb = 80
    nblk = n // rb
    nrb = -(-nblk // NS)

    @functools.partial(
        pl.kernel,
        out_type=jax.ShapeDtypeStruct((NC * n, 128), jnp.float32),
        mesh=_sc_mesh(),
        scratch_types=[
            pltpu.VMEM((ch,), jnp.int32),
            pltpu.VMEM((ch,), jnp.int32),
            pltpu.VMEM((ch, 128), jnp.float32),
            pltpu.VMEM((rb, 128), jnp.float32),
            pltpu.VMEM_SHARED((n, 128), jnp.float32),
            pltpu.SemaphoreType.DMA,
            pltpu.SemaphoreType.DMA,
        ],
    )
    def deg_kernel(dst_hbm, ones_hbm, zeros_hbm, out_hbm,
                   idxv0, idxv1, onesv, bounce, acc, sem0, sem1):
        c = lax.axis_index("c")
        s = lax.axis_index("s")
        idxv = (idxv0, idxv1)
        sem = (sem0, sem1)
        _zero_phase(zeros_hbm, bounce, acc, s, rb, nblk, nrb)
        pltpu.sync_copy(ones_hbm, onesv)
        plsc.subcore_barrier()
        e0 = c * (e // NC) + s * ncs * ch

        def pair(k, carry):
            for b in range(2):
                j = 2 * k + b

                @pl.when(k > 0)
                def _():
                    pltpu.make_async_copy(onesv, acc.at[idxv[b]], sem[b]).wait()

                pltpu.sync_copy(dst_hbm.at[pl.ds(e0 + j * ch, ch)], idxv[b])
                pltpu.async_copy(onesv, acc.at[idxv[b]], sem[b], add=True)
            return carry

        lax.fori_loop(0, npair, pair, 0)
        if rem:
            tbase = c * (e // NC) + (tot - rem) * ch

            @pl.when(s < rem)
            def _():
                pltpu.make_async_copy(onesv, acc.at[idxv[0]], sem[0]).wait()
                pltpu.sync_copy(dst_hbm.at[pl.ds(tbase + s * ch, ch)], idxv[0])
                pltpu.async_copy(onesv, acc.at[idxv[0]], sem[0], add=True)
        pltpu.make_async_copy(onesv, acc.at[idxv[0]], sem[0]).wait()
        pltpu.make_async_copy(onesv, acc.at[idxv[1]], sem[1]).wait()
        plsc.subcore_barrier()
        _copy_out_phase(acc, bounce, out_hbm, c, s, n, rb, nblk, nrb)

    return deg_kernel


def _make_spmm_edge(n, e, f):
    ch = 128
    tot = (e // NC) // ch
    ncs = tot // NS
    rem = tot % NS
    assert ncs % 2 == 0
    npair = ncs // 2
    rb = 80
    nblk = n // rb
    nrb = -(-nblk // NS)

    @functools.partial(
        pl.kernel,
        out_type=jax.ShapeDtypeStruct((NC * n, f), jnp.float32),
        mesh=_sc_mesh(),
        scratch_types=[
            pltpu.VMEM((ch,), jnp.int32),
            pltpu.VMEM((ch,), jnp.int32),
            pltpu.VMEM((ch,), jnp.int32),
            pltpu.VMEM((ch, f), jnp.float32),
            pltpu.VMEM((ch, f), jnp.float32),
            pltpu.VMEM((rb, f), jnp.float32),
            pltpu.VMEM_SHARED((n, f), jnp.float32),
            pltpu.SemaphoreType.DMA,
            pltpu.SemaphoreType.DMA,
        ],
    )
    def spmm_kernel(src_hbm, dst_hbm, g_hbm, zeros_hbm, out_hbm,
                    srcv, dstv0, dstv1, rows0, rows1, bounce, acc, sem0, sem1):
        c = lax.axis_index("c")
        s = lax.axis_index("s")
        dstv = (dstv0, dstv1)
        rows = (rows0, rows1)
        sem = (sem0, sem1)
        _zero_phase(zeros_hbm, bounce, acc, s, rb, nblk, nrb)
        plsc.subcore_barrier()
        e0 = c * (e // NC) + s * ncs * ch

        def pair(k, carry):
            for b in range(2):
                j = 2 * k + b
                base = e0 + j * ch
                pltpu.sync_copy(src_hbm.at[pl.ds(base, ch)], srcv)

                @pl.when(k > 0)
                def _():
                    pltpu.make_async_copy(rows[b], acc.at[dstv[b]], sem[b]).wait()

                pltpu.sync_copy(g_hbm.at[srcv], rows[b])
                pltpu.sync_copy(dst_hbm.at[pl.ds(base, ch)], dstv[b])
                pltpu.async_copy(rows[b], acc.at[dstv[b]], sem[b], add=True)
            return carry

        lax.fori_loop(0, npair, pair, 0)
        if rem:
            tbase = c * (e // NC) + (tot - rem) * ch

            @pl.when(s < rem)
            def _():
                base = tbase + s * ch
                pltpu.sync_copy(src_hbm.at[pl.ds(base, ch)], srcv)
                pltpu.make_async_copy(rows[0], acc.at[dstv[0]], sem[0]).wait()
                pltpu.sync_copy(g_hbm.at[srcv], rows[0])
                pltpu.sync_copy(dst_hbm.at[pl.ds(base, ch)], dstv[0])
                pltpu.async_copy(rows[0], acc.at[dstv[0]], sem[0], add=True)
        pltpu.make_async_copy(rows[0], acc.at[dstv[0]], sem[0]).wait()
        pltpu.make_async_copy(rows[1], acc.at[dstv[1]], sem[1]).wait()
        plsc.subcore_barrier()
        _copy_out_phase(acc, bounce, out_hbm, c, s, n, rb, nblk, nrb)

    return spmm_kernel


def _make_spmm_feat(n, e, fh):
    ch = 128
    tot = e // ch
    ncs = tot // NS
    rem = tot % NS
    assert ncs % 2 == 0
    npair = ncs // 2
    rb = 80
    nblk = n // rb
    nrb = -(-nblk // NS)

    @functools.partial(
        pl.kernel,
        out_type=jax.ShapeDtypeStruct((NC * n, fh), jnp.float32),
        mesh=_sc_mesh(),
        scratch_types=[
            pltpu.VMEM((ch,), jnp.int32),
            pltpu.VMEM((ch,), jnp.int32),
            pltpu.VMEM((ch,), jnp.int32),
            pltpu.VMEM((ch,), jnp.int32),
            pltpu.VMEM((ch, fh), jnp.float32),
            pltpu.VMEM((ch, fh), jnp.float32),
            pltpu.VMEM((rb, fh), jnp.float32),
            pltpu.VMEM_SHARED((n, fh), jnp.float32),
            pltpu.SemaphoreType.DMA,
            pltpu.SemaphoreType.DMA,
        ],
    )
    def spmm_kernel(src_hbm, dst_hbm, g_hbm, zeros_hbm, out_hbm,
                    srcv, idxg, dstv0, dstv1, rows0, rows1, bounce, acc,
                    sem0, sem1):
        c = lax.axis_index("c")
        s = lax.axis_index("s")
        dstv = (dstv0, dstv1)
        rows = (rows0, rows1)
        sem = (sem0, sem1)
        _zero_phase(zeros_hbm, bounce, acc, s, rb, nblk, nrb)
        plsc.subcore_barrier()
        e0 = s * ncs * ch

        def fill_idx():
            for kk in range(ch // 16):
                sl = pl.ds(kk * 16, 16)
                idxg[sl] = srcv[sl] * 2 + c

        def pair(k, carry):
            for b in range(2):
                j = 2 * k + b
                base = e0 + j * ch
                pltpu.sync_copy(src_hbm.at[pl.ds(base, ch)], srcv)
                fill_idx()

                @pl.when(k > 0)
                def _():
                    pltpu.make_async_copy(rows[b], acc.at[dstv[b]], sem[b]).wait()

                pltpu.sync_copy(g_hbm.at[idxg], rows[b])
                pltpu.sync_copy(dst_hbm.at[pl.ds(base, ch)], dstv[b])
                pltpu.async_copy(rows[b], acc.at[dstv[b]], sem[b], add=True)
            return carry

        lax.fori_loop(0, npair, pair, 0)
        if rem:
            tbase = (tot - rem) * ch

            @pl.when(s < rem)
            def _():
                base = tbase + s * ch
                pltpu.sync_copy(src_hbm.at[pl.ds(base, ch)], srcv)
                fill_idx()
                pltpu.make_async_copy(rows[0], acc.at[dstv[0]], sem[0]).wait()
                pltpu.sync_copy(g_hbm.at[idxg], rows[0])
                pltpu.sync_copy(dst_hbm.at[pl.ds(base, ch)], dstv[0])
                pltpu.async_copy(rows[0], acc.at[dstv[0]], sem[0], add=True)
        pltpu.make_async_copy(rows[0], acc.at[dstv[0]], sem[0]).wait()
        pltpu.make_async_copy(rows[1], acc.at[dstv[1]], sem[1]).wait()
        plsc.subcore_barrier()
        _copy_out_phase(acc, bounce, out_hbm, c, s, n, rb, nblk, nrb)

    return spmm_kernel


def _p1_kernel(dega_ref, degb_ref, x_ref, g1_ref, d16_ref):
    deg = dega_ref[...][:, :1] + degb_ref[...][:, :1] + 1.0
    d = lax.rsqrt(deg)
    g1_ref[...] = x_ref[...] * d
    d16_ref[...] = jnp.broadcast_to(d, d16_ref.shape)


def _p2_kernel(s1a_ref, s1b_ref, g1_ref, d16_ref, w1t_ref, b1_ref, g2_ref):
    d = d16_ref[...][:, :1]
    u = (s1a_ref[...] + s1b_ref[...] + g1_ref[...]) * d
    h = (jnp.dot(u, w1t_ref[...], preferred_element_type=jnp.float32)
         + b1_ref[...])
    g2_ref[...] = jnp.maximum(h, 0.0) * d


def _p3_kernel(s2a_ref, s2b_ref, g2_ref, d16_ref, w2at_ref, w2bt_ref, b2_ref,
               wf1t_ref, bf1_ref, wf2t_ref, bf2_ref, out_ref):
    d = d16_ref[...][:, :1]
    g2 = g2_ref[...]
    u = (s2a_ref[...] + g2[:, :128]) * d
    v = (s2b_ref[...] + g2[:, 128:]) * d
    h2 = jnp.maximum(
        jnp.dot(u, w2at_ref[...], preferred_element_type=jnp.float32)
        + jnp.dot(v, w2bt_ref[...], preferred_element_type=jnp.float32)
        + b2_ref[...], 0.0)
    h3 = jnp.maximum(
        jnp.dot(h2, wf1t_ref[...], preferred_element_type=jnp.float32)
        + bf1_ref[...], 0.0)
    out_ref[...] = (jnp.dot(h3, wf2t_ref[...], preferred_element_type=jnp.float32)
                    + bf2_ref[...])


def _row_spec(blk, width):
    return pl.BlockSpec((blk, width), lambda i: (i, 0))


def _full_spec(shape):
    return pl.BlockSpec(shape, lambda i: tuple(0 for _ in shape))


def kernel(x, edge_index, W1, b1, W2, b2, Wf1, bf1, Wf2, bf2):
    n, nfeat = x.shape
    e = edge_index.shape[1]
    nhid = W1.shape[0]
    blk = 1000
    grid = (n // blk,)

    src = edge_index[0]
    dst = edge_index[1]

    deg2 = _make_deg(n, e)(
        dst,
        jnp.ones((128, 128), jnp.float32),
        jnp.zeros((80, 128), jnp.float32),
    )

    g1, d16 = pl.pallas_call(
        _p1_kernel,
        grid=grid,
        in_specs=[_row_spec(blk, 128), _row_spec(blk, 128), _row_spec(blk, nfeat)],
        out_specs=[_row_spec(blk, nfeat), _row_spec(blk, 16)],
        out_shape=[
            jax.ShapeDtypeStruct((n, nfeat), jnp.float32),
            jax.ShapeDtypeStruct((n, 16), jnp.float32),
        ],
    )(deg2[:n], deg2[n:], x)

    zeros128a = jnp.zeros((80, nfeat), jnp.float32)
    s1 = _make_spmm_edge(n, e, nfeat)(src, dst, g1, zeros128a)

    g2 = pl.pallas_call(
        _p2_kernel,
        grid=grid,
        in_specs=[
            _row_spec(blk, nfeat), _row_spec(blk, nfeat), _row_spec(blk, nfeat),
            _row_spec(blk, 16),
            _full_spec((nfeat, nhid)), _full_spec((1, nhid)),
        ],
        out_specs=_row_spec(blk, nhid),
        out_shape=jax.ShapeDtypeStruct((n, nhid), jnp.float32),
    )(s1[:n], s1[n:], g1, d16, W1.T, b1.reshape(1, nhid))

    zeros128 = jnp.zeros((80, nhid // 2), jnp.float32)
    s2 = _make_spmm_feat(n, e, nhid // 2)(src, dst, g2.reshape(2 * n, nhid // 2),
                                          zeros128)

    out = pl.pallas_call(
        _p3_kernel,
        grid=grid,
        in_specs=[
            _row_spec(blk, 128), _row_spec(blk, 128), _row_spec(blk, nhid),
            _row_spec(blk, 16),
            _full_spec((128, nhid)), _full_spec((128, nhid)), _full_spec((1, nhid)),
            _full_spec((nhid, 128)), _full_spec((1, 128)),
            _full_spec((128, 16)), _full_spec((1, 16)),
        ],
        out_specs=_row_spec(blk, 16),
        out_shape=jax.ShapeDtypeStruct((n, 16), jnp.float32),
    )(s2[:n], s2[n:], g2, d16,
      W2[:, :128].T, W2[:, 128:].T, b2.reshape(1, nhid),
      Wf1.T, bf1.reshape(1, 128),
      Wf2.T, bf2.reshape(1, 16))
    return out


# speedup vs baseline: 2.2206x; 1.4426x over previous
import functools

import jax
import jax.numpy as jnp
from jax import lax
from jax.experimental import pallas as pl
from jax.experimental.pallas import tpu as pltpu
from jax.experimental.pallas import tpu_sc as plsc

NC = 2
NS = 16


def _sc_mesh():
    return plsc.VectorSubcoreMesh(
        core_axis_name="c", subcore_axis_name="s", num_cores=NC, num_subcores=NS
    )


def _zero_phase(zeros_hbm, bounce, acc, s, rb, nblk, nrb):
    pltpu.sync_copy(zeros_hbm, bounce)
    for j in range(nrb):
        bid = s * nrb + j

        @pl.when(bid < nblk)
        def _():
            pltpu.sync_copy(bounce, acc.at[pl.ds(bid * rb, rb)])


def _copy_out_phase(acc, bounce, out_hbm, c, s, n, rb, nblk, nrb):
    for j in range(nrb):
        bid = s * nrb + j

        @pl.when(bid < nblk)
        def _():
            pltpu.sync_copy(acc.at[pl.ds(bid * rb, rb)], bounce)
            pltpu.sync_copy(bounce, out_hbm.at[pl.ds(c * n + bid * rb, rb)])


def _make_deg(n, e):
    ch = 128
    tot = (e // NC) // ch
    ncs = tot // NS
    rem = tot % NS
    assert ncs % 2 == 0
    npair = ncs // 2
    rb = 80
    nblk = n // rb
    nrb = -(-nblk // NS)

    @functools.partial(
        pl.kernel,
        out_type=jax.ShapeDtypeStruct((NC * n, 128), jnp.float32),
        mesh=_sc_mesh(),
        scratch_types=[
            pltpu.VMEM((ch,), jnp.int32),
            pltpu.VMEM((ch,), jnp.int32),
            pltpu.VMEM((ch, 128), jnp.float32),
            pltpu.VMEM((rb, 128), jnp.float32),
            pltpu.VMEM_SHARED((n, 128), jnp.float32),
            pltpu.SemaphoreType.DMA,
            pltpu.SemaphoreType.DMA,
        ],
    )
    def deg_kernel(dst_hbm, ones_hbm, zeros_hbm, out_hbm,
                   idxv0, idxv1, onesv, bounce, acc, sem0, sem1):
        c = lax.axis_index("c")
        s = lax.axis_index("s")
        idxv = (idxv0, idxv1)
        sem = (sem0, sem1)
        _zero_phase(zeros_hbm, bounce, acc, s, rb, nblk, nrb)
        pltpu.sync_copy(ones_hbm, onesv)
        plsc.subcore_barrier()
        e0 = c * (e // NC) + s * ncs * ch

        def pair(k, carry):
            for b in range(2):
                j = 2 * k + b

                @pl.when(k > 0)
                def _():
                    pltpu.make_async_copy(onesv, acc.at[idxv[b]], sem[b]).wait()

                pltpu.sync_copy(dst_hbm.at[pl.ds(e0 + j * ch, ch)], idxv[b])
                pltpu.async_copy(onesv, acc.at[idxv[b]], sem[b], add=True)
            return carry

        lax.fori_loop(0, npair, pair, 0)
        if rem:
            tbase = c * (e // NC) + (tot - rem) * ch

            @pl.when(s < rem)
            def _():
                pltpu.make_async_copy(onesv, acc.at[idxv[0]], sem[0]).wait()
                pltpu.sync_copy(dst_hbm.at[pl.ds(tbase + s * ch, ch)], idxv[0])
                pltpu.async_copy(onesv, acc.at[idxv[0]], sem[0], add=True)
        pltpu.make_async_copy(onesv, acc.at[idxv[0]], sem[0]).wait()
        pltpu.make_async_copy(onesv, acc.at[idxv[1]], sem[1]).wait()
        plsc.subcore_barrier()
        _copy_out_phase(acc, bounce, out_hbm, c, s, n, rb, nblk, nrb)

    return deg_kernel


def _make_spmm_edge(n, e, f):
    ch = 128
    tot = (e // NC) // ch
    ncs = tot // NS
    rem = tot % NS
    assert ncs % 3 == 0
    rb = 80
    nblk = n // rb
    nrb = -(-nblk // NS)

    @functools.partial(
        pl.kernel,
        out_type=jax.ShapeDtypeStruct((NC * n, f), jnp.float32),
        mesh=_sc_mesh(),
        scratch_types=[
            [pltpu.VMEM((ch,), jnp.int32)] * 3,
            [pltpu.VMEM((ch,), jnp.int32)] * 3,
            [pltpu.VMEM((ch, f), jnp.float32)] * 3,
            pltpu.VMEM_SHARED((n, f), jnp.float32),
            [pltpu.SemaphoreType.DMA] * 3,
            [pltpu.SemaphoreType.DMA] * 3,
        ],
    )
    def spmm_kernel(src_hbm, dst_hbm, g_hbm, zeros_hbm, out_hbm,
                    srcv, dstv, rows, acc, gsem, ssem):
        c = lax.axis_index("c")
        s = lax.axis_index("s")
        bounce = rows[0].at[pl.ds(0, rb)]
        _zero_phase(zeros_hbm, bounce, acc, s, rb, nblk, nrb)
        plsc.subcore_barrier()
        e0 = c * (e // NC) + s * ncs * ch

        pltpu.sync_copy(src_hbm.at[pl.ds(e0, ch)], srcv[0])
        pltpu.async_copy(g_hbm.at[srcv[0]], rows[0], gsem[0])

        def triple(k, carry):
            for b in range(3):
                j = 3 * k + b
                bn = (b + 1) % 3
                base = e0 + j * ch
                pltpu.sync_copy(src_hbm.at[pl.ds(base + ch, ch)], srcv[bn])
                if b == 2:
                    pltpu.make_async_copy(rows[bn], acc.at[dstv[bn]],
                                          ssem[bn]).wait()
                else:
                    @pl.when(k > 0)
                    def _():
                        pltpu.make_async_copy(rows[bn], acc.at[dstv[bn]],
                                              ssem[bn]).wait()
                pltpu.async_copy(g_hbm.at[srcv[bn]], rows[bn], gsem[bn])
                pltpu.make_async_copy(g_hbm.at[srcv[b]], rows[b], gsem[b]).wait()
                pltpu.sync_copy(dst_hbm.at[pl.ds(base, ch)], dstv[b])
                pltpu.async_copy(rows[b], acc.at[dstv[b]], ssem[b], add=True)
            return carry

        lax.fori_loop(0, ncs // 3, triple, 0)
        pltpu.make_async_copy(g_hbm.at[srcv[0]], rows[0], gsem[0]).wait()
        pltpu.make_async_copy(rows[1], acc.at[dstv[1]], ssem[1]).wait()
        pltpu.make_async_copy(rows[2], acc.at[dstv[2]], ssem[2]).wait()
        if rem:
            tbase = c * (e // NC) + (tot - rem) * ch

            @pl.when(s < rem)
            def _():
                base = tbase + s * ch
                pltpu.sync_copy(src_hbm.at[pl.ds(base, ch)], srcv[0])
                pltpu.sync_copy(g_hbm.at[srcv[0]], rows[0])
                pltpu.sync_copy(dst_hbm.at[pl.ds(base, ch)], dstv[0])
                pltpu.sync_copy(rows[0], acc.at[dstv[0]], add=True)
        plsc.subcore_barrier()
        _copy_out_phase(acc, bounce, out_hbm, c, s, n, rb, nblk, nrb)

    return spmm_kernel


def _make_spmm_feat(n, e, fh):
    ch = 128
    tot = e // ch
    ncs = tot // NS
    rem = tot % NS
    assert ncs % 3 == 0
    rb = 80
    nblk = n // rb
    nrb = -(-nblk // NS)

    @functools.partial(
        pl.kernel,
        out_type=jax.ShapeDtypeStruct((NC * n, fh), jnp.float32),
        mesh=_sc_mesh(),
        scratch_types=[
            pltpu.VMEM((ch,), jnp.int32),
            [pltpu.VMEM((ch,), jnp.int32)] * 3,
            [pltpu.VMEM((ch,), jnp.int32)] * 3,
            [pltpu.VMEM((ch, fh), jnp.float32)] * 3,
            pltpu.VMEM_SHARED((n, fh), jnp.float32),
            [pltpu.SemaphoreType.DMA] * 3,
            [pltpu.SemaphoreType.DMA] * 3,
        ],
    )
    def spmm_kernel(src_hbm, dst_hbm, g_hbm, zeros_hbm, out_hbm,
                    srcv, idxg, dstv, rows, acc, gsem, ssem):
        c = lax.axis_index("c")
        s = lax.axis_index("s")
        bounce = rows[0].at[pl.ds(0, rb)]
        _zero_phase(zeros_hbm, bounce, acc, s, rb, nblk, nrb)
        plsc.subcore_barrier()
        e0 = s * ncs * ch

        def load_idx(base, buf):
            pltpu.sync_copy(src_hbm.at[pl.ds(base, ch)], srcv)
            for kk in range(ch // 16):
                sl = pl.ds(kk * 16, 16)
                idxg[buf][sl] = srcv[sl] * 2 + c

        load_idx(e0, 0)
        pltpu.async_copy(g_hbm.at[idxg[0]], rows[0], gsem[0])

        def triple(k, carry):
            for b in range(3):
                j = 3 * k + b
                bn = (b + 1) % 3
                base = e0 + j * ch
                load_idx(base + ch, bn)
                if b == 2:
                    pltpu.make_async_copy(rows[bn], acc.at[dstv[bn]],
                                          ssem[bn]).wait()
                else:
                    @pl.when(k > 0)
                    def _():
                        pltpu.make_async_copy(rows[bn], acc.at[dstv[bn]],
                                              ssem[bn]).wait()
                pltpu.async_copy(g_hbm.at[idxg[bn]], rows[bn], gsem[bn])
                pltpu.make_async_copy(g_hbm.at[idxg[b]], rows[b], gsem[b]).wait()
                pltpu.sync_copy(dst_hbm.at[pl.ds(base, ch)], dstv[b])
                pltpu.async_copy(rows[b], acc.at[dstv[b]], ssem[b], add=True)
            return carry

        lax.fori_loop(0, ncs // 3, triple, 0)
        pltpu.make_async_copy(g_hbm.at[idxg[0]], rows[0], gsem[0]).wait()
        pltpu.make_async_copy(rows[1], acc.at[dstv[1]], ssem[1]).wait()
        pltpu.make_async_copy(rows[2], acc.at[dstv[2]], ssem[2]).wait()
        if rem:
            tbase = (tot - rem) * ch

            @pl.when(s < rem)
            def _():
                base = tbase + s * ch
                load_idx(base, 0)
                pltpu.sync_copy(g_hbm.at[idxg[0]], rows[0])
                pltpu.sync_copy(dst_hbm.at[pl.ds(base, ch)], dstv[0])
                pltpu.sync_copy(rows[0], acc.at[dstv[0]], add=True)
        plsc.subcore_barrier()
        _copy_out_phase(acc, bounce, out_hbm, c, s, n, rb, nblk, nrb)

    return spmm_kernel


def _p1_kernel(dega_ref, degb_ref, x_ref, g1_ref, d16_ref):
    deg = dega_ref[...][:, :1] + degb_ref[...][:, :1] + 1.0
    d = lax.rsqrt(deg)
    g1_ref[...] = x_ref[...] * d
    d16_ref[...] = jnp.broadcast_to(d, d16_ref.shape)


def _p2_kernel(s1a_ref, s1b_ref, g1_ref, d16_ref, w1t_ref, b1_ref, g2_ref):
    d = d16_ref[...][:, :1]
    u = (s1a_ref[...] + s1b_ref[...] + g1_ref[...]) * d
    h = (jnp.dot(u, w1t_ref[...], preferred_element_type=jnp.float32)
         + b1_ref[...])
    g2_ref[...] = jnp.maximum(h, 0.0) * d


def _p3_kernel(s2a_ref, s2b_ref, g2_ref, d16_ref, w2at_ref, w2bt_ref, b2_ref,
               wf1t_ref, bf1_ref, wf2t_ref, bf2_ref, out_ref):
    d = d16_ref[...][:, :1]
    g2 = g2_ref[...]
    u = (s2a_ref[...] + g2[:, :128]) * d
    v = (s2b_ref[...] + g2[:, 128:]) * d
    h2 = jnp.maximum(
        jnp.dot(u, w2at_ref[...], preferred_element_type=jnp.float32)
        + jnp.dot(v, w2bt_ref[...], preferred_element_type=jnp.float32)
        + b2_ref[...], 0.0)
    h3 = jnp.maximum(
        jnp.dot(h2, wf1t_ref[...], preferred_element_type=jnp.float32)
        + bf1_ref[...], 0.0)
    out_ref[...] = (jnp.dot(h3, wf2t_ref[...], preferred_element_type=jnp.float32)
                    + bf2_ref[...])


def _row_spec(blk, width):
    return pl.BlockSpec((blk, width), lambda i: (i, 0))


def _full_spec(shape):
    return pl.BlockSpec(shape, lambda i: tuple(0 for _ in shape))


def kernel(x, edge_index, W1, b1, W2, b2, Wf1, bf1, Wf2, bf2):
    n, nfeat = x.shape
    e = edge_index.shape[1]
    nhid = W1.shape[0]
    blk = 1000
    grid = (n // blk,)

    src = edge_index[0]
    dst = edge_index[1]

    deg2 = _make_deg(n, e)(
        dst,
        jnp.ones((128, 128), jnp.float32),
        jnp.zeros((80, 128), jnp.float32),
    )

    g1, d16 = pl.pallas_call(
        _p1_kernel,
        grid=grid,
        in_specs=[_row_spec(blk, 128), _row_spec(blk, 128), _row_spec(blk, nfeat)],
        out_specs=[_row_spec(blk, nfeat), _row_spec(blk, 16)],
        out_shape=[
            jax.ShapeDtypeStruct((n, nfeat), jnp.float32),
            jax.ShapeDtypeStruct((n, 16), jnp.float32),
        ],
    )(deg2[:n], deg2[n:], x)

    zeros128a = jnp.zeros((80, nfeat), jnp.float32)
    s1 = _make_spmm_edge(n, e, nfeat)(src, dst, g1, zeros128a)

    g2 = pl.pallas_call(
        _p2_kernel,
        grid=grid,
        in_specs=[
            _row_spec(blk, nfeat), _row_spec(blk, nfeat), _row_spec(blk, nfeat),
            _row_spec(blk, 16),
            _full_spec((nfeat, nhid)), _full_spec((1, nhid)),
        ],
        out_specs=_row_spec(blk, nhid),
        out_shape=jax.ShapeDtypeStruct((n, nhid), jnp.float32),
    )(s1[:n], s1[n:], g1, d16, W1.T, b1.reshape(1, nhid))

    zeros128 = jnp.zeros((80, nhid // 2), jnp.float32)
    s2 = _make_spmm_feat(n, e, nhid // 2)(src, dst, g2.reshape(2 * n, nhid // 2),
                                          zeros128)

    out = pl.pallas_call(
        _p3_kernel,
        grid=grid,
        in_specs=[
            _row_spec(blk, 128), _row_spec(blk, 128), _row_spec(blk, nhid),
            _row_spec(blk, 16),
            _full_spec((128, nhid)), _full_spec((128, nhid)), _full_spec((1, nhid)),
            _full_spec((nhid, 128)), _full_spec((1, 128)),
            _full_spec((128, 16)), _full_spec((1, 16)),
        ],
        out_specs=_row_spec(blk, 16),
        out_shape=jax.ShapeDtypeStruct((n, 16), jnp.float32),
    )(s2[:n], s2[n:], g2, d16,
      W2[:, :128].T, W2[:, 128:].T, b2.reshape(1, nhid),
      Wf1.T, bf1.reshape(1, 128),
      Wf2.T, bf2.reshape(1, 16))
    return out


# speedup vs baseline: 2.2306x; 1.0045x over previous
import functools

import jax
import jax.numpy as jnp
from jax import lax
from jax.experimental import pallas as pl
from jax.experimental.pallas import tpu as pltpu
from jax.experimental.pallas import tpu_sc as plsc

NC = 2
NS = 16


def _sc_mesh():
    return plsc.VectorSubcoreMesh(
        core_axis_name="c", subcore_axis_name="s", num_cores=NC, num_subcores=NS
    )


def _zero_phase(zeros_hbm, bounce, acc, s, rb, nblk, nrb):
    pltpu.sync_copy(zeros_hbm, bounce)
    for j in range(nrb):
        bid = s * nrb + j

        @pl.when(bid < nblk)
        def _():
            pltpu.sync_copy(bounce, acc.at[pl.ds(bid * rb, rb)])


def _copy_out_phase(acc, bounce, out_hbm, c, s, n, rb, nblk, nrb):
    for j in range(nrb):
        bid = s * nrb + j

        @pl.when(bid < nblk)
        def _():
            pltpu.sync_copy(acc.at[pl.ds(bid * rb, rb)], bounce)
            pltpu.sync_copy(bounce, out_hbm.at[pl.ds(c * n + bid * rb, rb)])


def _make_deg(n, e, w=128):
    ch = 128
    tot = (e // NC) // ch
    ncs = tot // NS
    rem = tot % NS
    assert ncs % 3 == 0
    rb = 80
    nblk = n // rb
    nrb = -(-nblk // NS)

    @functools.partial(
        pl.kernel,
        out_type=jax.ShapeDtypeStruct((NC * n, w), jnp.float32),
        mesh=_sc_mesh(),
        scratch_types=[
            [pltpu.VMEM((ch,), jnp.int32)] * 3,
            pltpu.VMEM((ch, w), jnp.float32),
            pltpu.VMEM((rb, w), jnp.float32),
            pltpu.VMEM_SHARED((n, w), jnp.float32),
            [pltpu.SemaphoreType.DMA] * 3,
        ],
    )
    def deg_kernel(dst_hbm, ones_hbm, zeros_hbm, out_hbm,
                   idxv, onesv, bounce, acc, sem):
        c = lax.axis_index("c")
        s = lax.axis_index("s")
        _zero_phase(zeros_hbm, bounce, acc, s, rb, nblk, nrb)
        pltpu.sync_copy(ones_hbm, onesv)
        plsc.subcore_barrier()
        e0 = c * (e // NC) + s * ncs * ch

        def triple(k, carry):
            for b in range(3):
                j = 3 * k + b

                @pl.when(k > 0)
                def _():
                    pltpu.make_async_copy(onesv, acc.at[idxv[b]], sem[b]).wait()

                pltpu.sync_copy(dst_hbm.at[pl.ds(e0 + j * ch, ch)], idxv[b])
                pltpu.async_copy(onesv, acc.at[idxv[b]], sem[b], add=True)
            return carry

        lax.fori_loop(0, ncs // 3, triple, 0)
        if rem:
            tbase = c * (e // NC) + (tot - rem) * ch

            @pl.when(s < rem)
            def _():
                pltpu.make_async_copy(onesv, acc.at[idxv[0]], sem[0]).wait()
                pltpu.sync_copy(dst_hbm.at[pl.ds(tbase + s * ch, ch)], idxv[0])
                pltpu.async_copy(onesv, acc.at[idxv[0]], sem[0], add=True)
        pltpu.make_async_copy(onesv, acc.at[idxv[0]], sem[0]).wait()
        pltpu.make_async_copy(onesv, acc.at[idxv[1]], sem[1]).wait()
        pltpu.make_async_copy(onesv, acc.at[idxv[2]], sem[2]).wait()
        plsc.subcore_barrier()
        _copy_out_phase(acc, bounce, out_hbm, c, s, n, rb, nblk, nrb)

    return deg_kernel


def _make_spmm_edge(n, e, f):
    ch = 128
    tot = (e // NC) // ch
    ncs = tot // NS
    rem = tot % NS
    assert ncs % 3 == 0
    rb = 80
    nblk = n // rb
    nrb = -(-nblk // NS)

    @functools.partial(
        pl.kernel,
        out_type=jax.ShapeDtypeStruct((NC * n, f), jnp.float32),
        mesh=_sc_mesh(),
        scratch_types=[
            [pltpu.VMEM((ch,), jnp.int32)] * 3,
            [pltpu.VMEM((ch,), jnp.int32)] * 3,
            [pltpu.VMEM((ch, f), jnp.float32)] * 3,
            pltpu.VMEM_SHARED((n, f), jnp.float32),
            [pltpu.SemaphoreType.DMA] * 3,
            [pltpu.SemaphoreType.DMA] * 3,
        ],
    )
    def spmm_kernel(src_hbm, dst_hbm, g_hbm, zeros_hbm, out_hbm,
                    srcv, dstv, rows, acc, gsem, ssem):
        c = lax.axis_index("c")
        s = lax.axis_index("s")
        bounce = rows[0].at[pl.ds(0, rb)]
        _zero_phase(zeros_hbm, bounce, acc, s, rb, nblk, nrb)
        plsc.subcore_barrier()
        e0 = c * (e // NC) + s * ncs * ch

        pltpu.sync_copy(src_hbm.at[pl.ds(e0, ch)], srcv[0])
        pltpu.async_copy(g_hbm.at[srcv[0]], rows[0], gsem[0])

        def triple(k, carry):
            for b in range(3):
                j = 3 * k + b
                bn = (b + 1) % 3
                base = e0 + j * ch
                pltpu.sync_copy(src_hbm.at[pl.ds(base + ch, ch)], srcv[bn])
                if b == 2:
                    pltpu.make_async_copy(rows[bn], acc.at[dstv[bn]],
                                          ssem[bn]).wait()
                else:
                    @pl.when(k > 0)
                    def _():
                        pltpu.make_async_copy(rows[bn], acc.at[dstv[bn]],
                                              ssem[bn]).wait()
                pltpu.async_copy(g_hbm.at[srcv[bn]], rows[bn], gsem[bn])
                pltpu.make_async_copy(g_hbm.at[srcv[b]], rows[b], gsem[b]).wait()
                pltpu.sync_copy(dst_hbm.at[pl.ds(base, ch)], dstv[b])
                pltpu.async_copy(rows[b], acc.at[dstv[b]], ssem[b], add=True)
            return carry

        lax.fori_loop(0, ncs // 3, triple, 0)
        pltpu.make_async_copy(g_hbm.at[srcv[0]], rows[0], gsem[0]).wait()
        pltpu.make_async_copy(rows[1], acc.at[dstv[1]], ssem[1]).wait()
        pltpu.make_async_copy(rows[2], acc.at[dstv[2]], ssem[2]).wait()
        if rem:
            tbase = c * (e // NC) + (tot - rem) * ch

            @pl.when(s < rem)
            def _():
                base = tbase + s * ch
                pltpu.sync_copy(src_hbm.at[pl.ds(base, ch)], srcv[0])
                pltpu.sync_copy(g_hbm.at[srcv[0]], rows[0])
                pltpu.sync_copy(dst_hbm.at[pl.ds(base, ch)], dstv[0])
                pltpu.sync_copy(rows[0], acc.at[dstv[0]], add=True)
        plsc.subcore_barrier()
        _copy_out_phase(acc, bounce, out_hbm, c, s, n, rb, nblk, nrb)

    return spmm_kernel


def _make_spmm_feat(n, e, fh):
    ch = 128
    tot = e // ch
    ncs = tot // NS
    rem = tot % NS
    assert ncs % 3 == 0
    rb = 80
    nblk = n // rb
    nrb = -(-nblk // NS)

    @functools.partial(
        pl.kernel,
        out_type=jax.ShapeDtypeStruct((NC * n, fh), jnp.float32),
        mesh=_sc_mesh(),
        scratch_types=[
            pltpu.VMEM((ch,), jnp.int32),
            [pltpu.VMEM((ch,), jnp.int32)] * 3,
            [pltpu.VMEM((ch,), jnp.int32)] * 3,
            [pltpu.VMEM((ch, fh), jnp.float32)] * 3,
            pltpu.VMEM_SHARED((n, fh), jnp.float32),
            [pltpu.SemaphoreType.DMA] * 3,
            [pltpu.SemaphoreType.DMA] * 3,
        ],
    )
    def spmm_kernel(src_hbm, dst_hbm, g_hbm, zeros_hbm, out_hbm,
                    srcv, idxg, dstv, rows, acc, gsem, ssem):
        c = lax.axis_index("c")
        s = lax.axis_index("s")
        bounce = rows[0].at[pl.ds(0, rb)]
        _zero_phase(zeros_hbm, bounce, acc, s, rb, nblk, nrb)
        plsc.subcore_barrier()
        e0 = s * ncs * ch

        def load_idx(base, buf):
            pltpu.sync_copy(src_hbm.at[pl.ds(base, ch)], srcv)
            for kk in range(ch // 16):
                sl = pl.ds(kk * 16, 16)
                idxg[buf][sl] = srcv[sl] * 2 + c

        load_idx(e0, 0)
        pltpu.async_copy(g_hbm.at[idxg[0]], rows[0], gsem[0])

        def triple(k, carry):
            for b in range(3):
                j = 3 * k + b
                bn = (b + 1) % 3
                base = e0 + j * ch
                load_idx(base + ch, bn)
                if b == 2:
                    pltpu.make_async_copy(rows[bn], acc.at[dstv[bn]],
                                          ssem[bn]).wait()
                else:
                    @pl.when(k > 0)
                    def _():
                        pltpu.make_async_copy(rows[bn], acc.at[dstv[bn]],
                                              ssem[bn]).wait()
                pltpu.async_copy(g_hbm.at[idxg[bn]], rows[bn], gsem[bn])
                pltpu.make_async_copy(g_hbm.at[idxg[b]], rows[b], gsem[b]).wait()
                pltpu.sync_copy(dst_hbm.at[pl.ds(base, ch)], dstv[b])
                pltpu.async_copy(rows[b], acc.at[dstv[b]], ssem[b], add=True)
            return carry

        lax.fori_loop(0, ncs // 3, triple, 0)
        pltpu.make_async_copy(g_hbm.at[idxg[0]], rows[0], gsem[0]).wait()
        pltpu.make_async_copy(rows[1], acc.at[dstv[1]], ssem[1]).wait()
        pltpu.make_async_copy(rows[2], acc.at[dstv[2]], ssem[2]).wait()
        if rem:
            tbase = (tot - rem) * ch

            @pl.when(s < rem)
            def _():
                base = tbase + s * ch
                load_idx(base, 0)
                pltpu.sync_copy(g_hbm.at[idxg[0]], rows[0])
                pltpu.sync_copy(dst_hbm.at[pl.ds(base, ch)], dstv[0])
                pltpu.sync_copy(rows[0], acc.at[dstv[0]], add=True)
        plsc.subcore_barrier()
        _copy_out_phase(acc, bounce, out_hbm, c, s, n, rb, nblk, nrb)

    return spmm_kernel


def _p1_kernel(dega_ref, degb_ref, x_ref, g1_ref, d16_ref):
    deg = dega_ref[...][:, :1] + degb_ref[...][:, :1] + 1.0
    d = lax.rsqrt(deg)
    g1_ref[...] = x_ref[...] * d
    d16_ref[...] = jnp.broadcast_to(d, d16_ref.shape)


def _p2_kernel(s1a_ref, s1b_ref, g1_ref, d16_ref, w1t_ref, b1_ref, g2_ref):
    d = d16_ref[...][:, :1]
    u = (s1a_ref[...] + s1b_ref[...] + g1_ref[...]) * d
    h = (jnp.dot(u, w1t_ref[...], preferred_element_type=jnp.float32)
         + b1_ref[...])
    g2_ref[...] = jnp.maximum(h, 0.0) * d


def _p3_kernel(s2a_ref, s2b_ref, g2_ref, d16_ref, w2at_ref, w2bt_ref, b2_ref,
               wf1t_ref, bf1_ref, wf2t_ref, bf2_ref, out_ref):
    d = d16_ref[...][:, :1]
    g2 = g2_ref[...]
    u = (s2a_ref[...] + g2[:, :128]) * d
    v = (s2b_ref[...] + g2[:, 128:]) * d
    h2 = jnp.maximum(
        jnp.dot(u, w2at_ref[...], preferred_element_type=jnp.float32)
        + jnp.dot(v, w2bt_ref[...], preferred_element_type=jnp.float32)
        + b2_ref[...], 0.0)
    h3 = jnp.maximum(
        jnp.dot(h2, wf1t_ref[...], preferred_element_type=jnp.float32)
        + bf1_ref[...], 0.0)
    out_ref[...] = (jnp.dot(h3, wf2t_ref[...], preferred_element_type=jnp.float32)
                    + bf2_ref[...])


def _row_spec(blk, width):
    return pl.BlockSpec((blk, width), lambda i: (i, 0))


def _full_spec(shape):
    return pl.BlockSpec(shape, lambda i: tuple(0 for _ in shape))


def kernel(x, edge_index, W1, b1, W2, b2, Wf1, bf1, Wf2, bf2):
    n, nfeat = x.shape
    e = edge_index.shape[1]
    nhid = W1.shape[0]
    blk = 1000
    grid = (n // blk,)

    src = edge_index[0]
    dst = edge_index[1]

    deg2 = _make_deg(n, e)(
        dst,
        jnp.ones((128, 128), jnp.float32),
        jnp.zeros((80, 128), jnp.float32),
    )

    g1, d16 = pl.pallas_call(
        _p1_kernel,
        grid=grid,
        in_specs=[_row_spec(blk, 128), _row_spec(blk, 128), _row_spec(blk, nfeat)],
        out_specs=[_row_spec(blk, nfeat), _row_spec(blk, 16)],
        out_shape=[
            jax.ShapeDtypeStruct((n, nfeat), jnp.float32),
            jax.ShapeDtypeStruct((n, 16), jnp.float32),
        ],
    )(deg2[:n], deg2[n:], x)

    zeros128a = jnp.zeros((80, nfeat), jnp.float32)
    s1 = _make_spmm_edge(n, e, nfeat)(src, dst, g1, zeros128a)

    g2 = pl.pallas_call(
        _p2_kernel,
        grid=grid,
        in_specs=[
            _row_spec(blk, nfeat), _row_spec(blk, nfeat), _row_spec(blk, nfeat),
            _row_spec(blk, 16),
            _full_spec((nfeat, nhid)), _full_spec((1, nhid)),
        ],
        out_specs=_row_spec(blk, nhid),
        out_shape=jax.ShapeDtypeStruct((n, nhid), jnp.float32),
    )(s1[:n], s1[n:], g1, d16, W1.T, b1.reshape(1, nhid))

    zeros128 = jnp.zeros((80, nhid // 2), jnp.float32)
    s2 = _make_spmm_feat(n, e, nhid // 2)(src, dst, g2.reshape(2 * n, nhid // 2),
                                          zeros128)

    out = pl.pallas_call(
        _p3_kernel,
        grid=grid,
        in_specs=[
            _row_spec(blk, 128), _row_spec(blk, 128), _row_spec(blk, nhid),
            _row_spec(blk, 16),
            _full_spec((128, nhid)), _full_spec((128, nhid)), _full_spec((1, nhid)),
            _full_spec((nhid, 128)), _full_spec((1, 128)),
            _full_spec((128, 16)), _full_spec((1, 16)),
        ],
        out_specs=_row_spec(blk, 16),
        out_shape=jax.ShapeDtypeStruct((n, 16), jnp.float32),
    )(s2[:n], s2[n:], g2, d16,
      W2[:, :128].T, W2[:, 128:].T, b2.reshape(1, nhid),
      Wf1.T, bf1.reshape(1, 128),
      Wf2.T, bf2.reshape(1, 16))
    return out
